# Initial kernel scaffold; baseline (speedup 1.0000x reference)
import jax
import jax.numpy as jnp
from jax.experimental import pallas as pl


def kernel(x):
    raise NotImplementedError("write your pallas kernel here")


# speedup vs baseline: 54.8595x; 54.8595x over previous
import jax
import jax.numpy as jnp
from jax import lax
from jax.experimental import pallas as pl
from jax.experimental.pallas import tpu as pltpu
from jax.experimental.pallas import tpu_sc as plsc

B, N, C = 128, 8192, 32
NW = 32
BPW = B // NW
PIECE = 1024
NPIECE = N // PIECE
BLK = 32
NBLK = PIECE // BLK
K = 9
NCHUNKS = BPW * NPIECE
PELEMS = PIECE * C

_F32_NEG_INF = float("-inf")


def _splat_f(val):
    return jnp.full((16,), val, dtype=jnp.float32)


def _splat_i(val):
    return jnp.full((16,), val, dtype=jnp.int32)


def _argmax_merge(v1, r1, v2, r2):
    upd = v2 > v1
    return jnp.where(upd, v2, v1), jnp.where(upd, r2, r1)


def _sc_body(x_hbm, out_hbm, buf0, buf1, blockmax, blockrow, cand, outstage,
             sem0, sem1):
    wid = lax.axis_index("s") * 2 + lax.axis_index("c")
    iota16 = lax.iota(jnp.int32, 16)
    neginf = _splat_f(_F32_NEG_INF)

    def chunk_src(chunk):
        b = wid * BPW + chunk // NPIECE
        piece = chunk % NPIECE
        return x_hbm.at[b, pl.ds(piece * PELEMS, PELEMS)]

    def process(cur, chunk):
        b = wid * BPW + chunk // NPIECE
        piece = chunk % NPIECE
        for g in range(2):
            lane = _splat_i(g * 16) + iota16

            def blockbody(blk, _):
                base = blk * (BLK * C) + g * 16
                accs = []
                for a in range(4):
                    accs.append((cur[pl.ds(base + a * C, 16)],
                                 _splat_i(a * C) + lane))
                for j in range(4, BLK):
                    a = j % 4
                    v = cur[pl.ds(base + j * C, 16)]
                    av, ar = accs[a]
                    upd = v > av
                    accs[a] = (jnp.where(upd, v, av),
                               jnp.where(upd, _splat_i(j * C) + lane, ar))
                m01 = _argmax_merge(*accs[0], *accs[1])
                m23 = _argmax_merge(*accs[2], *accs[3])
                bm, br = _argmax_merge(*m01, *m23)
                blockmax[pl.ds(blk * 16, 16)] = bm
                blockrow[pl.ds(blk * 16, 16)] = br + _splat_i(blk * BLK * C)
                return _

            lax.fori_loop(0, NBLK, blockbody, 0)

            def iterbody(i, _):
                accs = []
                for a in range(4):
                    accs.append((blockmax[pl.ds(a * 16, 16)], _splat_i(a)))
                for j in range(4, NBLK):
                    a = j % 4
                    v = blockmax[pl.ds(j * 16, 16)]
                    av, ar = accs[a]
                    upd = v > av
                    accs[a] = (jnp.where(upd, v, av),
                               jnp.where(upd, _splat_i(j), ar))
                m01 = _argmax_merge(*accs[0], *accs[1])
                m23 = _argmax_merge(*accs[2], *accs[3])
                val, bidx = _argmax_merge(*m01, *m23)

                pos = plsc.load_gather(blockrow, [bidx * 16 + iota16])
                plsc.store_scatter(cur, [pos], neginf)

                rbase = bidx * (BLK * C) + lane
                raccs = []
                for a in range(4):
                    ridx = rbase + _splat_i(a * C)
                    raccs.append((plsc.load_gather(cur, [ridx]), ridx))
                for j in range(4, BLK):
                    ridx = rbase + _splat_i(j * C)
                    rv = plsc.load_gather(cur, [ridx])
                    a = j % 4
                    av, ar = raccs[a]
                    upd = rv > av
                    raccs[a] = (jnp.where(upd, rv, av),
                                jnp.where(upd, ridx, ar))
                r01 = _argmax_merge(*raccs[0], *raccs[1])
                r23 = _argmax_merge(*raccs[2], *raccs[3])
                nbm, nbr = _argmax_merge(*r01, *r23)
                plsc.store_scatter(blockmax, [bidx * 16 + iota16], nbm)
                plsc.store_scatter(blockrow, [bidx * 16 + iota16], nbr)

                cand[pl.ds((g * 80 + piece * K + i) * 16, 16)] = val
                return _

            lax.fori_loop(0, K, iterbody, 0)

            @pl.when(piece == NPIECE - 1)
            def _finalize():
                cbase = g * 80 * 16

                def fbody(i, s):
                    accs = []
                    for a in range(4):
                        accs.append((cand[pl.ds(cbase + a * 16, 16)],
                                     _splat_i(cbase + a * 16) + iota16))
                    for j in range(4, NPIECE * K):
                        a = j % 4
                        v = cand[pl.ds(cbase + j * 16, 16)]
                        av, ar = accs[a]
                        upd = v > av
                        accs[a] = (jnp.where(upd, v, av),
                                   jnp.where(upd,
                                             _splat_i(cbase + j * 16) + iota16,
                                             ar))
                    m01 = _argmax_merge(*accs[0], *accs[1])
                    m23 = _argmax_merge(*accs[2], *accs[3])
                    bv, bpos = _argmax_merge(*m01, *m23)
                    plsc.store_scatter(cand, [bpos], neginf)
                    return s + bv

                sums = lax.fori_loop(0, K, fbody, _splat_f(0.0))
                outstage[:] = sums * (1.0 / K)
                pltpu.sync_copy(outstage, out_hbm.at[b, pl.ds(g * 16, 16)])

    pltpu.async_copy(chunk_src(0), buf0, sem0)

    def outer(k, _):
        for par in range(2):
            chunk = 2 * k + par
            cur = buf0 if par == 0 else buf1
            nxt = buf1 if par == 0 else buf0
            cursem = sem0 if par == 0 else sem1
            nxtsem = sem1 if par == 0 else sem0
            pltpu.make_async_copy(chunk_src(0), cur, cursem).wait()

            @pl.when(chunk < NCHUNKS - 1)
            def _start_next():
                pltpu.async_copy(chunk_src(chunk + 1), nxt, nxtsem)

            process(cur, chunk)
        return _

    lax.fori_loop(0, NCHUNKS // 2, outer, 0)


@jax.jit
def _topk_mean(x):
    mesh = plsc.VectorSubcoreMesh(core_axis_name="c", subcore_axis_name="s")
    f = pl.kernel(
        _sc_body,
        out_type=jax.ShapeDtypeStruct((B, C), jnp.float32),
        mesh=mesh,
        compiler_params=pltpu.CompilerParams(needs_layout_passes=False),
        scratch_types=[
            pltpu.VMEM((PELEMS,), jnp.float32),
            pltpu.VMEM((PELEMS,), jnp.float32),
            pltpu.VMEM((NBLK * 16,), jnp.float32),
            pltpu.VMEM((NBLK * 16,), jnp.int32),
            pltpu.VMEM((160 * 16,), jnp.float32),
            pltpu.VMEM((16,), jnp.float32),
            pltpu.SemaphoreType.DMA,
            pltpu.SemaphoreType.DMA,
        ],
    )
    return f(x.reshape(B, N * C))


def kernel(x):
    return _topk_mean(x)
